# baseline (speedup 1.0000x reference)
import functools

import jax
import jax.numpy as jnp
from jax import lax
from jax.experimental import pallas as pl
from jax.experimental.pallas import tpu as pltpu
from jax.experimental.pallas import tpu_sc as plsc

F32 = jnp.float32

N_ATOMS = 50000
N_BONDS = 800000
MAX_NB = 16
FA = 133
FB = 147
H = 100
HP = 128
N_MOLS = 2500
APM = 20
DEPTH = 3

NC = 2
NS = 16
NW = NC * NS

BA = 400
NAP = 51200
NBLK_A = NAP // BA
BLK_A_PER_W = NBLK_A // NW

BB = 200
NBLK_B = N_BONDS // BB
BLK_B_PER_W = NBLK_B // NW

BM = 1600
BMA = 2000

_SC_MESH = plsc.VectorSubcoreMesh(core_axis_name="c", subcore_axis_name="s")


def _wid():
    return lax.axis_index("s") * NC + lax.axis_index("c")


@functools.partial(
    pl.kernel,
    out_type=jax.ShapeDtypeStruct((NAP, HP), F32),
    mesh=_SC_MESH,
    scratch_types=[
        pltpu.VMEM((2 * MAX_NB * BA,), jnp.int32),
        pltpu.VMEM((BA, HP), F32),
        pltpu.VMEM((BA, HP), F32),
    ] + [pltpu.SemaphoreType.DMA] * 6,
)
def _sc_segsum(m_hbm, a2bb_hbm, out_hbm, idx_v, acc0, acc1,
               si0, si1, sg0, sg1, sw0, sw1):
    wid = _wid()
    first = wid * BLK_A_PER_W
    SI, SG, SW, ACC = (si0, si1), (sg0, sg1), (sw0, sw1), (acc0, acc1)
    CHUNK = MAX_NB * BA

    def iref(p, k):
        return idx_v.at[pl.ds(p * CHUNK + k * BA, BA)]

    def stage(j, p):
        pltpu.async_copy(
            a2bb_hbm.at[pl.ds((first + j) * CHUNK, CHUNK)],
            idx_v.at[pl.ds(p * CHUNK, CHUNK)], SI[p])

    def wait_idx(p):
        pltpu.make_async_copy(
            a2bb_hbm.at[pl.ds(0, CHUNK)],
            idx_v.at[pl.ds(p * CHUNK, CHUNK)], SI[p]).wait()

    def fire0(p):
        pltpu.async_copy(m_hbm.at[iref(p, 0)], ACC[p], SG[p])

    def wait_g(p, n):
        for _ in range(n):
            pltpu.make_async_copy(m_hbm.at[iref(p, 0)], ACC[p], SG[p]).wait()

    def fire_adds(p):
        for k in range(1, MAX_NB):
            pltpu.async_copy(m_hbm.at[iref(p, k)], ACC[p], SG[p], add=True)

    def fire_write(j, p):
        pltpu.async_copy(ACC[p], out_hbm.at[pl.ds((first + j) * BA, BA)], SW[p])

    def wait_write(p):
        pltpu.make_async_copy(ACC[p], out_hbm.at[pl.ds(0, BA)], SW[p]).wait()

    stage(0, 0)
    stage(1, 1)
    wait_idx(0)
    fire0(0)
    for j in range(BLK_A_PER_W):
        p, q = j % 2, (j + 1) % 2
        wait_g(p, 1)
        fire_adds(p)
        if j + 1 < BLK_A_PER_W:
            if j >= 1:
                wait_write(q)
            wait_idx(q)
            fire0(q)
        wait_g(p, MAX_NB - 1)
        fire_write(j, p)
        if j + 2 < BLK_A_PER_W:
            stage(j + 2, p)
    wait_write(0)
    wait_write(1)


@functools.partial(
    pl.kernel,
    out_type=jax.ShapeDtypeStruct((N_BONDS, HP), F32),
    mesh=_SC_MESH,
    scratch_types=[
        pltpu.VMEM((2 * BB,), jnp.int32),
        pltpu.VMEM((2 * BB,), jnp.int32),
        pltpu.VMEM((BB, HP), F32),
        pltpu.VMEM((BB, HP), F32),
        pltpu.VMEM((BB, HP), F32),
        pltpu.VMEM((BB, HP), F32),
    ] + [pltpu.SemaphoreType.DMA] * 6,
)
def _sc_diff(a_hbm, m_hbm, b2a_hbm, b2revb_hbm, out_hbm,
             ia_v, ib_v, a0, a1, m0, m1, si0, si1, sg0, sg1, sw0, sw1):
    wid = _wid()
    first = wid * BLK_B_PER_W
    SI, SG, SW = (si0, si1), (sg0, sg1), (sw0, sw1)
    BUFA, BUFM = (a0, a1), (m0, m1)

    def islices(p):
        return ia_v.at[pl.ds(p * BB, BB)], ib_v.at[pl.ds(p * BB, BB)]

    def stage(j, p):
        base = (first + j) * BB
        ia, ib = islices(p)
        pltpu.async_copy(b2a_hbm.at[pl.ds(base, BB)], ia, SI[p])
        pltpu.async_copy(b2revb_hbm.at[pl.ds(base, BB)], ib, SI[p])

    def fire(p):
        ia, ib = islices(p)
        pltpu.make_async_copy(b2a_hbm.at[pl.ds(0, BB)], ia, SI[p]).wait()
        pltpu.make_async_copy(b2revb_hbm.at[pl.ds(0, BB)], ib, SI[p]).wait()
        pltpu.async_copy(a_hbm.at[ia], BUFA[p], SG[p])
        pltpu.async_copy(m_hbm.at[ib], BUFM[p], SG[p])

    def wait_write(p):
        pltpu.make_async_copy(BUFA[p], out_hbm.at[pl.ds(0, BB)], SW[p]).wait()

    def finish(j, p):
        ia, ib = islices(p)
        pltpu.make_async_copy(a_hbm.at[ia], BUFA[p], SG[p]).wait()
        pltpu.make_async_copy(m_hbm.at[ib], BUFM[p], SG[p]).wait()
        ba, bm = BUFA[p], BUFM[p]

        def sub_row(r, carry):
            for c in range(HP // 16):
                sl = pl.ds(c * 16, 16)
                ba[r, sl] = ba[r, sl] - bm[r, sl]
            return carry

        lax.fori_loop(0, BB, sub_row, 0)
        pltpu.async_copy(ba, out_hbm.at[pl.ds((first + j) * BB, BB)], SW[p])

    def body(j, p, q, do_ww, do_fire, do_stage):
        if do_ww:
            wait_write(q)
        if do_fire:
            fire(q)
        finish(j, p)
        if do_stage:
            stage(j + 2, p)

    stage(0, 0)
    stage(1, 1)
    fire(0)
    body(0, 0, 1, False, True, True)

    def pair(t, carry):
        j = 2 * t + 1
        body(j, 1, 0, True, True, True)
        body(j + 1, 0, 1, True, True, True)
        return carry

    lax.fori_loop(0, (BLK_B_PER_W - 3) // 2, pair, 0)
    body(BLK_B_PER_W - 2, 1, 0, True, True, False)
    body(BLK_B_PER_W - 1, 0, 1, True, False, False)
    wait_write(0)


def _t1_body(x_ref, w_ref, inp_ref, m0_ref):
    y = lax.dot_general(x_ref[...], w_ref[...], (((1,), (1,)), ((), ())),
                        preferred_element_type=F32)
    inp_ref[...] = y
    m0_ref[...] = jnp.maximum(y, 0.0)


def _tc_input(f_bonds, wi_p):
    return pl.pallas_call(
        _t1_body,
        grid=(N_BONDS // BM,),
        in_specs=[
            pl.BlockSpec((BM, FB), lambda i: (i, 0)),
            pl.BlockSpec((HP, FB), lambda i: (0, 0)),
        ],
        out_specs=[
            pl.BlockSpec((BM, HP), lambda i: (i, 0)),
            pl.BlockSpec((BM, HP), lambda i: (i, 0)),
        ],
        out_shape=[
            jax.ShapeDtypeStruct((N_BONDS, HP), F32),
            jax.ShapeDtypeStruct((N_BONDS, HP), F32),
        ],
    )(f_bonds, wi_p)


def _t2_body(inp_ref, d_ref, w_ref, out_ref):
    y = lax.dot_general(d_ref[...], w_ref[...], (((1,), (1,)), ((), ())),
                        preferred_element_type=F32)
    out_ref[...] = jnp.maximum(inp_ref[...] + y, 0.0)


def _tc_update(inp, diff, wh_p):
    return pl.pallas_call(
        _t2_body,
        grid=(N_BONDS // BM,),
        in_specs=[
            pl.BlockSpec((BM, HP), lambda i: (i, 0)),
            pl.BlockSpec((BM, HP), lambda i: (i, 0)),
            pl.BlockSpec((HP, HP), lambda i: (0, 0)),
        ],
        out_specs=pl.BlockSpec((BM, HP), lambda i: (i, 0)),
        out_shape=jax.ShapeDtypeStruct((N_BONDS, HP), F32),
    )(inp, diff, wh_p)


def _t4_body(fa_ref, a_ref, woa_ref, wom_ref, st_ref, ah_ref, mol_ref):
    y = lax.dot_general(fa_ref[...], woa_ref[...], (((1,), (1,)), ((), ())),
                        preferred_element_type=F32)
    y = y + lax.dot_general(a_ref[:, :H], wom_ref[...], (((1,), (1,)), ((), ())),
                            preferred_element_type=F32)
    ah = jnp.maximum(y, 0.0)
    ah_ref[...] = ah
    mol_ref[...] = lax.dot_general(st_ref[...], ah, (((1,), (0,)), ((), ())),
                                   preferred_element_type=F32)[None]


def _tc_readout(f_atoms, a_msg, wo_a, wo_m, st):
    return pl.pallas_call(
        _t4_body,
        grid=(N_ATOMS // BMA,),
        in_specs=[
            pl.BlockSpec((BMA, FA), lambda i: (i, 0)),
            pl.BlockSpec((BMA, HP), lambda i: (i, 0)),
            pl.BlockSpec((H, FA), lambda i: (0, 0)),
            pl.BlockSpec((H, H), lambda i: (0, 0)),
            pl.BlockSpec((BMA // APM, BMA), lambda i: (0, 0)),
        ],
        out_specs=[
            pl.BlockSpec((BMA, H), lambda i: (i, 0)),
            pl.BlockSpec((1, BMA // APM, H), lambda i: (i, 0, 0)),
        ],
        out_shape=[
            jax.ShapeDtypeStruct((N_ATOMS, H), F32),
            jax.ShapeDtypeStruct((N_ATOMS // BMA, BMA // APM, H), F32),
        ],
    )(f_atoms, a_msg, wo_a, wo_m, st)


def kernel(f_atoms, f_bonds, f_mol, W_i, W_h, W_o, a2b, b2a, b2revb, ascope):
    wi_p = jnp.zeros((HP, FB), F32).at[:H].set(W_i)
    wh_p = jnp.zeros((HP, HP), F32).at[:H, :H].set(W_h)
    wo_a = W_o[:, :FA]
    wo_m = W_o[:, FA:]
    a2bp = jnp.zeros((NAP, MAX_NB), jnp.int32).at[:N_ATOMS].set(
        a2b.astype(jnp.int32))
    a2bb = a2bp.T.reshape(MAX_NB, NBLK_A, BA).transpose(1, 0, 2).reshape(-1)
    b2a32 = b2a.astype(jnp.int32)
    b2revb32 = b2revb.astype(jnp.int32)

    inp, msg = _tc_input(f_bonds, wi_p)
    for _ in range(DEPTH - 1):
        a_msg = _sc_segsum(msg, a2bb)
        diff = _sc_diff(a_msg, msg, b2a32, b2revb32)
        msg = _tc_update(inp, diff, wh_p)
    a_msg = _sc_segsum(msg, a2bb)

    st = jnp.repeat(jnp.eye(BMA // APM, dtype=F32), APM, axis=1)
    atom_hiddens, mol_sum = _tc_readout(f_atoms, a_msg, wo_a, wo_m, st)
    mol_sum = mol_sum.reshape(N_MOLS, H)
    sizes = ascope[:, 1].astype(F32)
    mol_vecs = jnp.concatenate([mol_sum / sizes[:, None], f_mol], axis=1)
    return (mol_vecs, atom_hiddens)
